# baseline (speedup 1.0000x reference)
import jax
import jax.numpy as jnp
from jax.experimental import pallas as pl

_MARGIN = 1.35
_VB = 20000


def _scale_body(lab_ref, x_ref, o_ref):
    i = pl.program_id(0)
    x = x_ref[...]
    rows = jax.lax.broadcasted_iota(jnp.int32, x.shape, 0) + i * _VB
    o_ref[...] = jnp.where(rows == lab_ref[...], x * _MARGIN, x)


def kernel(logits, label):
    b, v = logits.shape
    xt = logits.T
    lab = label.astype(jnp.int32).reshape(1, b)
    out_t = pl.pallas_call(
        _scale_body,
        grid=(pl.cdiv(v, _VB),),
        in_specs=[
            pl.BlockSpec((1, b), lambda i: (0, 0)),
            pl.BlockSpec((_VB, b), lambda i: (i, 0)),
        ],
        out_specs=pl.BlockSpec((_VB, b), lambda i: (i, 0)),
        out_shape=jax.ShapeDtypeStruct((v, b), logits.dtype),
    )(lab, xt)
    return out_t.T


# speedup vs baseline: 1.0214x; 1.0214x over previous
import jax
import jax.numpy as jnp
from jax.experimental import pallas as pl

_MARGIN = 1.35
_VB = 25000


def _scale_body(lab_ref, x_ref, o_ref):
    i = pl.program_id(0)
    x = x_ref[...]
    rows = jax.lax.broadcasted_iota(jnp.int32, x.shape, 0) + i * _VB
    o_ref[...] = jnp.where(rows == lab_ref[...], x * _MARGIN, x)


def kernel(logits, label):
    b, v = logits.shape
    xt = logits.T
    lab = label.astype(jnp.int32).reshape(1, b)
    out_t = pl.pallas_call(
        _scale_body,
        grid=(pl.cdiv(v, _VB),),
        in_specs=[
            pl.BlockSpec((1, b), lambda i: (0, 0)),
            pl.BlockSpec((_VB, b), lambda i: (i, 0)),
        ],
        out_specs=pl.BlockSpec((_VB, b), lambda i: (i, 0)),
        out_shape=jax.ShapeDtypeStruct((v, b), logits.dtype),
    )(lab, xt)
    return out_t.T


# speedup vs baseline: 1.0669x; 1.0446x over previous
import jax
import jax.numpy as jnp
from jax.experimental import pallas as pl

_MARGIN = 1.35
_VB = 28000


def _scale_body(lab_ref, x_ref, o_ref):
    i = pl.program_id(0)
    x = x_ref[...]
    rows = jax.lax.broadcasted_iota(jnp.int32, x.shape, 0) + i * _VB
    o_ref[...] = jnp.where(rows == lab_ref[...], x * _MARGIN, x)


def kernel(logits, label):
    b, v = logits.shape
    xt = logits.T
    lab = label.astype(jnp.int32).reshape(1, b)
    out_t = pl.pallas_call(
        _scale_body,
        grid=(pl.cdiv(v, _VB),),
        in_specs=[
            pl.BlockSpec((1, b), lambda i: (0, 0)),
            pl.BlockSpec((_VB, b), lambda i: (i, 0)),
        ],
        out_specs=pl.BlockSpec((_VB, b), lambda i: (i, 0)),
        out_shape=jax.ShapeDtypeStruct((v, b), logits.dtype),
    )(lab, xt)
    return out_t.T


# speedup vs baseline: 1.0724x; 1.0051x over previous
import jax
import jax.numpy as jnp
from jax.experimental import pallas as pl

_MARGIN = 1.35
_VB = 29992


def _scale_body(lab_ref, x_ref, o_ref):
    i = pl.program_id(0)
    x = x_ref[...]
    rows = jax.lax.broadcasted_iota(jnp.int32, x.shape, 0) + i * _VB
    o_ref[...] = jnp.where(rows == lab_ref[...], x * _MARGIN, x)


def kernel(logits, label):
    b, v = logits.shape
    xt = logits.T
    lab = label.astype(jnp.int32).reshape(1, b)
    out_t = pl.pallas_call(
        _scale_body,
        grid=(pl.cdiv(v, _VB),),
        in_specs=[
            pl.BlockSpec((1, b), lambda i: (0, 0)),
            pl.BlockSpec((_VB, b), lambda i: (i, 0)),
        ],
        out_specs=pl.BlockSpec((_VB, b), lambda i: (i, 0)),
        out_shape=jax.ShapeDtypeStruct((v, b), logits.dtype),
    )(lab, xt)
    return out_t.T
